# baseline (speedup 1.0000x reference)
import functools
import math

import jax
import jax.numpy as jnp
from jax import lax
from jax.experimental import pallas as pl
from jax.experimental.pallas import tpu as pltpu
from jax.experimental.pallas import tpu_sc as plsc

N = 10000
E = 160000
IN_DIM = 128
D1 = 128
D2 = 64
NC = 2
MB = 1000

NW = 32
E_PAD = 163840
PW = E_PAD // NW
CH = 128
NCHUNK = PW // CH
NPAD = 10240
DV = 32
NEG = -1e30


def _elu(x):
    return jnp.where(x > 0, x, jnp.exp(x) - 1.0)


def _proj_body(d, x_ref, w_ref, b_ref, o_ref, qbf_ref, kbf_ref):
    res = (
        jnp.dot(x_ref[...], w_ref[...], preferred_element_type=jnp.float32)
        + b_ref[...]
    )
    o_ref[...] = res
    qbf_ref[...] = res[:, :d].astype(jnp.bfloat16)
    kbf_ref[...] = res[:, d:2 * d].astype(jnp.bfloat16)


def _proj(x, wcat, bcat, d):
    n, din = x.shape
    dout = wcat.shape[1]
    return pl.pallas_call(
        functools.partial(_proj_body, d),
        grid=(n // MB,),
        in_specs=[
            pl.BlockSpec((MB, din), lambda i: (i, 0)),
            pl.BlockSpec((din, dout), lambda i: (0, 0)),
            pl.BlockSpec((1, dout), lambda i: (0, 0)),
        ],
        out_specs=[pl.BlockSpec((MB, dout), lambda i: (i, 0)),
                   pl.BlockSpec((MB, d), lambda i: (i, 0)),
                   pl.BlockSpec((MB, d), lambda i: (i, 0))],
        out_shape=[jax.ShapeDtypeStruct((n, dout), jnp.float32),
                   jax.ShapeDtypeStruct((n, d), jnp.bfloat16),
                   jax.ShapeDtypeStruct((n, d), jnp.bfloat16)],
    )(x, wcat, bcat.reshape(1, dout))


def _edge_body(d, q_hbm, k_hbm, v0_hbm, v1_hbm, v2_hbm, v3_hbm,
               srcr_hbm, dstr_hbm,
               agg_hbm, den_hbm, m_hbm,
               qa, qb, ka, kb, va, vb, lbuf, srcb, dstb, mbuf, mxv,
               zbuf, zdbuf, agg_sh, den_sh, mxs_sh, sema, semb, semc):
    c = lax.axis_index("c")
    s = lax.axis_index("s")
    w = s * 2 + c
    base = w * PW
    inv = 1.0 / math.sqrt(d)
    nsl = d // 16
    nh = d // DV
    lane = jnp.arange(16, dtype=jnp.int32)
    base_row = s * 640

    pltpu.sync_copy(srcr_hbm.at[w], srcb)
    pltpu.sync_copy(dstr_hbm.at[w], dstb)

    def zrow(i, carry):
        zbuf[i % 40, pl.ds((i // 40) * 16, 16)] = jnp.zeros((16,), jnp.float32)
        return carry

    lax.fori_loop(0, 40 * DV // 16, zrow, 0)

    def zden(i, carry):
        zdbuf[pl.ds(i * 16, 16)] = jnp.zeros((16,), jnp.float32)
        return carry

    lax.fori_loop(0, 40, zden, 0)
    for t in range(16):
        pltpu.sync_copy(zbuf, agg_sh.at[pl.ds(base_row + t * 40, 40)])
    pltpu.sync_copy(zdbuf, den_sh.at[pl.ds(s * 640, 640)])

    def _dot_chunk(j, qrow, krow):
        def group(g, carry2):
            lgv = jnp.full((16,), NEG, jnp.float32)
            for p in range(16):
                i = g * 16 + p
                acc = None
                for cc in range(nsl // 2):
                    qv = qrow[i, pl.ds(cc * 32, 32)]
                    kv = krow[i, pl.ds(cc * 32, 32)]
                    q0, q1 = plsc.unpack(qv, format=plsc.PackFormat.INTERLEAVED)
                    k0, k1 = plsc.unpack(kv, format=plsc.PackFormat.INTERLEAVED)
                    t = q0 * k0 + q1 * k1
                    acc = t if acc is None else acc + t
                lg = jnp.sum(acc) * inv
                lgv = jnp.where(lane == p, jnp.full((16,), lg, jnp.float32), lgv)
            eid = base + j * CH + g * 16 + lane
            lgv = jnp.where(eid < E, lgv, jnp.full((16,), NEG, jnp.float32))
            lbuf[pl.ds(j * CH + g * 16, 16)] = lgv
            return carry2

        lax.fori_loop(0, CH // 16, group, 0)

    def _start_qk(j, qrow, krow, sem):
        pltpu.async_copy(q_hbm.at[dstb.at[j]], qrow, sem)
        pltpu.async_copy(k_hbm.at[srcb.at[j]], krow, sem)

    def _wait_qk(qrow, krow, sem):
        pltpu.make_async_copy(q_hbm.at[dstb.at[0]], qrow, sem).wait()
        pltpu.make_async_copy(k_hbm.at[srcb.at[0]], krow, sem).wait()

    _start_qk(0, qa, ka, sema)

    def chunkpair(t, carry):
        j0 = 2 * t
        j1 = 2 * t + 1
        _start_qk(j1, qb, kb, semb)
        _wait_qk(qa, ka, sema)
        _dot_chunk(j0, qa, ka)
        _start_qk(jnp.minimum(j0 + 2, NCHUNK - 1), qa, ka, sema)
        _wait_qk(qb, kb, semb)
        _dot_chunk(j1, qb, kb)
        return carry

    lax.fori_loop(0, NCHUNK // 2, chunkpair, 0)
    _wait_qk(qa, ka, sema)

    def mxstep(i, m):
        return jnp.maximum(m, lbuf[pl.ds(i * 16, 16)])

    wmax = lax.fori_loop(0, PW // 16, mxstep, jnp.full((16,), NEG, jnp.float32))
    mbuf[0, pl.ds(0, 16)] = wmax
    pltpu.sync_copy(mbuf, mxs_sh.at[pl.ds(s, 1)])
    plsc.subcore_barrier()
    pltpu.sync_copy(mxs_sh, mxv)

    def gmx(i, m):
        return jnp.maximum(m, mxv[i, pl.ds(0, 16)])

    gv = lax.fori_loop(0, 16, gmx, jnp.full((16,), NEG, jnp.float32))
    mc = jnp.max(gv)
    mcv = jnp.full((16,), mc, jnp.float32)

    @pl.when(s == 0)
    def _dump_m():
        mbuf[0, pl.ds(0, 16)] = mcv
        pltpu.sync_copy(mbuf, m_hbm.at[c])

    def estep(i, carry):
        lbuf[pl.ds(i * 16, 16)] = jnp.exp(lbuf[pl.ds(i * 16, 16)] - mcv)
        return carry

    lax.fori_loop(0, PW // 16, estep, 0)

    for h in range(nh):
        vh_hbm = (v0_hbm, v1_hbm, v2_hbm, v3_hbm)[h]

        def _scale(j, vrow):
            def scale(g, carry2):
                ev = lbuf[pl.ds(j * CH + g * 16, 16)]
                for p in range(16):
                    i = g * 16 + p
                    bc = jnp.full((16,), ev[p], jnp.float32)
                    for cc in range(DV // 16):
                        vrow[i, pl.ds(cc * 16, 16)] = (
                            vrow[i, pl.ds(cc * 16, 16)] * bc)
                return carry2

            lax.fori_loop(0, CH // 16, scale, 0)

        def _wait_v(vrow, sem, vh_hbm=vh_hbm):
            pltpu.make_async_copy(vh_hbm.at[srcb.at[0]], vrow, sem).wait()

        pltpu.async_copy(vh_hbm.at[srcb.at[0]], va, sema)

        def chunk2pair(t, carry, vh_hbm=vh_hbm, h=h):
            j0 = 2 * t
            j1 = 2 * t + 1
            pltpu.async_copy(vh_hbm.at[srcb.at[j1]], vb, semb)
            _wait_v(va, sema)
            _scale(j0, va)
            if h == 0:
                pltpu.sync_copy(lbuf.at[pl.ds(j0 * CH, CH)],
                                den_sh.at[dstb.at[j0]], add=True)
            pltpu.async_copy(va, agg_sh.at[dstb.at[j0]], semc, add=True)
            _wait_v(vb, semb)
            _scale(j1, vb)
            if h == 0:
                pltpu.sync_copy(lbuf.at[pl.ds(j1 * CH, CH)],
                                den_sh.at[dstb.at[j1]], add=True)
            pltpu.make_async_copy(va, agg_sh.at[dstb.at[j0]], semc).wait()
            pltpu.async_copy(vh_hbm.at[srcb.at[jnp.minimum(j0 + 2, NCHUNK - 1)]],
                             va, sema)
            pltpu.sync_copy(vb, agg_sh.at[dstb.at[j1]], add=True)
            return carry

        lax.fori_loop(0, NCHUNK // 2, chunk2pair, 0)
        _wait_v(va, sema)
        plsc.subcore_barrier()

        for t in range(5):
            pltpu.sync_copy(agg_sh.at[pl.ds(base_row + t * 128, 128)],
                            agg_hbm.at[c, h, pl.ds(base_row + t * 128, 128)])
        if h == 0:
            pltpu.sync_copy(den_sh.at[pl.ds(s * 640, 640)],
                            den_hbm.at[c, pl.ds(s * 640, 640)])
        if h + 1 < nh:
            for t in range(16):
                pltpu.sync_copy(zbuf, agg_sh.at[pl.ds(base_row + t * 40, 40)])
            plsc.subcore_barrier()


def _make_edge(d):
    mesh = plsc.VectorSubcoreMesh(core_axis_name="c", subcore_axis_name="s")
    return pl.kernel(
        functools.partial(_edge_body, d),
        out_type=[
            jax.ShapeDtypeStruct((2, d // DV, NPAD, DV), jnp.float32),
            jax.ShapeDtypeStruct((2, NPAD), jnp.float32),
            jax.ShapeDtypeStruct((2, 1, 16), jnp.float32),
        ],
        mesh=mesh,
        name=f"edge_d{d}",
        compiler_params=pltpu.CompilerParams(
            needs_layout_passes=False, use_tc_tiling_on_sc=False),
        scratch_types=[
            pltpu.VMEM((CH, d), jnp.bfloat16),
            pltpu.VMEM((CH, d), jnp.bfloat16),
            pltpu.VMEM((CH, d), jnp.bfloat16),
            pltpu.VMEM((CH, d), jnp.bfloat16),
            pltpu.VMEM((CH, DV), jnp.float32),
            pltpu.VMEM((CH, DV), jnp.float32),
            pltpu.VMEM((PW,), jnp.float32),
            pltpu.VMEM((NCHUNK, CH), jnp.int32),
            pltpu.VMEM((NCHUNK, CH), jnp.int32),
            pltpu.VMEM((1, 16), jnp.float32),
            pltpu.VMEM((16, 16), jnp.float32),
            pltpu.VMEM((40, DV), jnp.float32),
            pltpu.VMEM((640,), jnp.float32),
            pltpu.VMEM_SHARED((NPAD, DV), jnp.float32),
            pltpu.VMEM_SHARED((NPAD,), jnp.float32),
            pltpu.VMEM_SHARED((16, 16), jnp.float32),
            pltpu.SemaphoreType.DMA,
            pltpu.SemaphoreType.DMA,
            pltpu.SemaphoreType.DMA,
        ],
    )


_EDGE = {d: _make_edge(d) for d in (D1, D2)}


def _edges(qbf, kbf, v, srcr, dstr, d):
    vs = [v[:, h * DV:(h + 1) * DV] for h in range(d // DV)]
    while len(vs) < 4:
        vs.append(vs[0])
    agg_p, den_p, m_p = _EDGE[d](qbf, kbf, *vs, srcr, dstr)
    return agg_p, den_p.reshape(2, NPAD, 1), m_p


def _rescale(m_ref):
    mv = m_ref[...]
    mm = jnp.max(mv)
    sv = jnp.exp(mv - mm)
    return sv[0, 0, 0], sv[1, 0, 0]


def _glue_body(agg_ref, denp_ref, m_ref, s_ref, w_ref, b_ref, o_ref,
               qbf_ref, kbf_ref):
    s0, s1 = _rescale(m_ref)
    agg = (jnp.concatenate([agg_ref[0, h] for h in range(4)], axis=1) * s0
           + jnp.concatenate([agg_ref[1, h] for h in range(4)], axis=1) * s1)
    den = denp_ref[0] * s0 + denp_ref[1] * s1
    h1 = _elu(agg / (den + 1e-16) + s_ref[...])
    res = (
        jnp.dot(h1, w_ref[...], preferred_element_type=jnp.float32) + b_ref[...]
    )
    o_ref[...] = res
    qbf_ref[...] = res[:, :D2].astype(jnp.bfloat16)
    kbf_ref[...] = res[:, D2:2 * D2].astype(jnp.bfloat16)


def _glue(agg_p, den_p, m_p, skip, wcat, bcat):
    n, din = skip.shape
    dout = wcat.shape[1]
    return pl.pallas_call(
        _glue_body,
        grid=(n // MB,),
        in_specs=[
            pl.BlockSpec((2, 4, MB, DV), lambda i: (0, 0, i, 0)),
            pl.BlockSpec((2, MB, 1), lambda i: (0, i, 0)),
            pl.BlockSpec((2, 1, 16), lambda i: (0, 0, 0)),
            pl.BlockSpec((MB, din), lambda i: (i, 0)),
            pl.BlockSpec((din, dout), lambda i: (0, 0)),
            pl.BlockSpec((1, dout), lambda i: (0, 0)),
        ],
        out_specs=[pl.BlockSpec((MB, dout), lambda i: (i, 0)),
                   pl.BlockSpec((MB, D2), lambda i: (i, 0)),
                   pl.BlockSpec((MB, D2), lambda i: (i, 0))],
        out_shape=[jax.ShapeDtypeStruct((n, dout), jnp.float32),
                   jax.ShapeDtypeStruct((n, D2), jnp.bfloat16),
                   jax.ShapeDtypeStruct((n, D2), jnp.bfloat16)],
    )(agg_p, den_p, m_p, skip, wcat, bcat.reshape(1, dout))


def _mlp3(h, w1, b1, w2, b2, w3, b3):
    h = jax.nn.relu(jnp.dot(h, w1, preferred_element_type=jnp.float32) + b1)
    h = jax.nn.relu(jnp.dot(h, w2, preferred_element_type=jnp.float32) + b2)
    return jnp.dot(h, w3, preferred_element_type=jnp.float32) + b3


def _head_body(agg_ref, denp_ref, m_ref, s_ref, wb_ref, bb_ref, wfc_ref,
               bfc_ref, wd1_ref, bd1_ref, wd2_ref, bd2_ref, wd3_ref, bd3_ref,
               wc1_ref, bc1_ref, wc2_ref, bc2_ref, wc3_ref, bc3_ref,
               emb_ref, dom_ref, cls0_ref, cls1_ref):
    s0, s1 = _rescale(m_ref)
    agg = (jnp.concatenate([agg_ref[0, 0], agg_ref[0, 1]], axis=1) * s0
           + jnp.concatenate([agg_ref[1, 0], agg_ref[1, 1]], axis=1) * s1)
    den = denp_ref[0] * s0 + denp_ref[1] * s1
    h2 = _elu(agg / (den + 1e-16) + s_ref[...])
    ss = jnp.dot(h2, wb_ref[...], preferred_element_type=jnp.float32) + bb_ref[...]
    emb = jnp.dot(ss, wfc_ref[...], preferred_element_type=jnp.float32) + bfc_ref[...]
    emb_ref[...] = emb
    m = jnp.max(emb, axis=1, keepdims=True)
    ex = jnp.exp(emb - m)
    p = ex / jnp.sum(ex, axis=1, keepdims=True)
    dom_ref[...] = _mlp3(ss, wd1_ref[...], bd1_ref[...], wd2_ref[...],
                         bd2_ref[...], wd3_ref[...], bd3_ref[...])
    for i, cls_ref in ((0, cls0_ref), (1, cls1_ref)):
        f = p[:, i][:, None] * ss
        cls_ref[...] = _mlp3(f, wc1_ref[i], bc1_ref[i], wc2_ref[i],
                             bc2_ref[i], wc3_ref[i], bc3_ref[i])


def _heads(agg_p, den_p, m_p, s2, Wb, bb, Wfc, bfc, Wd1, bd1, Wd2, bd2,
           Wd3, bd3, Wc1, bc1, Wc2, bc2, Wc3, bc3):
    n = s2.shape[0]
    full = lambda *shape: pl.BlockSpec(shape, lambda i: (0,) * len(shape))
    row = lambda d: pl.BlockSpec((MB, d), lambda i: (i, 0))
    out_shapes = [jax.ShapeDtypeStruct((n, 2), jnp.float32)] * 4
    return pl.pallas_call(
        _head_body,
        grid=(n // MB,),
        in_specs=[
            pl.BlockSpec((2, 2, MB, DV), lambda i: (0, 0, i, 0)),
            pl.BlockSpec((2, MB, 1), lambda i: (0, i, 0)),
            pl.BlockSpec((2, 1, 16), lambda i: (0, 0, 0)),
            row(D2),
            full(D2, 32), full(1, 32), full(32, NC), full(1, NC),
            full(32, 512), full(1, 512), full(512, 512), full(1, 512),
            full(512, 2), full(1, 2),
            full(NC, 32, 512), full(NC, 1, 512),
            full(NC, 512, 512), full(NC, 1, 512),
            full(NC, 512, 2), full(NC, 1, 2),
        ],
        out_specs=[row(2), row(2), row(2), row(2)],
        out_shape=out_shapes,
    )(agg_p, den_p, m_p, s2, Wb, bb.reshape(1, 32), Wfc, bfc.reshape(1, NC),
      Wd1, bd1.reshape(1, 512), Wd2, bd2.reshape(1, 512),
      Wd3, bd3.reshape(1, 2),
      Wc1, bc1.reshape(NC, 1, 512), Wc2, bc2.reshape(NC, 1, 512),
      Wc3, bc3.reshape(NC, 1, 2))


def _pad_edges(ei):
    pad = jnp.arange(E_PAD - E, dtype=jnp.int32) % N
    src = jnp.concatenate([ei[0], pad]).reshape(NW, NCHUNK, CH)
    dst = jnp.concatenate([ei[1], pad]).reshape(NW, NCHUNK, CH)
    return src, dst


def kernel(source_x, target_x, source_edge_index, target_edge_index,
           Wq1, bq1, Wk1, bk1, Wv1, bv1, Ws1, bs1,
           Wq2, bq2, Wk2, bk2, Wv2, bv2, Ws2, bs2,
           Wb, bb, Wfc, bfc,
           Wd1, bd1, Wd2, bd2, Wd3, bd3,
           Wc1, bc1, Wc2, bc2, Wc3, bc3):
    wcat1 = jnp.concatenate([Wq1, Wk1, Wv1, Ws1], axis=1)
    bcat1 = jnp.concatenate([bq1, bk1, bv1, bs1])
    wcat2 = jnp.concatenate([Wq2, Wk2, Wv2, Ws2], axis=1)
    bcat2 = jnp.concatenate([bq2, bk2, bv2, bs2])

    def share(x, ei):
        srcr, dstr = _pad_edges(ei)
        qkvs1, qbf1, kbf1 = _proj(x, wcat1, bcat1, D1)
        agg1, den1, m1 = _edges(qbf1, kbf1,
                                qkvs1[:, 2 * D1:3 * D1], srcr, dstr, D1)
        qkvs2, qbf2, kbf2 = _glue(agg1, den1, m1, qkvs1[:, 3 * D1:],
                                  wcat2, bcat2)
        agg2, den2, m2 = _edges(qbf2, kbf2,
                                qkvs2[:, 2 * D2:3 * D2], srcr, dstr, D2)
        return agg2, den2, m2, qkvs2[:, 3 * D2:]

    s_agg2, s_den2, s_m2, s_s2 = share(source_x, source_edge_index)
    t_agg2, t_den2, t_m2, t_s2 = share(target_x, target_edge_index)

    head_w = (Wb, bb, Wfc, bfc, Wd1, bd1, Wd2, bd2, Wd3, bd3,
              Wc1, bc1, Wc2, bc2, Wc3, bc3)
    s_emb, s_dom, s_cls0, s_cls1 = _heads(s_agg2, s_den2, s_m2, s_s2, *head_w)
    _, t_dom, t_cls0, t_cls1 = _heads(t_agg2, t_den2, t_m2, t_s2, *head_w)

    s_out = jnp.stack([s_cls0, s_cls1])
    t_out = jnp.stack([t_cls0, t_cls1])
    return (s_emb, s_dom, t_dom, s_out, t_out)


# speedup vs baseline: 1.0202x; 1.0202x over previous
import functools
import math

import jax
import jax.numpy as jnp
from jax import lax
from jax.experimental import pallas as pl
from jax.experimental.pallas import tpu as pltpu
from jax.experimental.pallas import tpu_sc as plsc

N = 10000
E = 160000
IN_DIM = 128
D1 = 128
D2 = 64
NC = 2
MB = 1000

NW = 32
E_PAD = 163840
PW = E_PAD // NW
CH = 128
NCHUNK = PW // CH
NPAD = 10240
DV = 32
NEG = -1e30


def _elu(x):
    return jnp.where(x > 0, x, jnp.exp(x) - 1.0)


def _proj_body(d, x_ref, w_ref, b_ref, o_ref, qbf_ref, kbf_ref):
    res = (
        jnp.dot(x_ref[...], w_ref[...], preferred_element_type=jnp.float32)
        + b_ref[...]
    )
    o_ref[...] = res
    qbf_ref[...] = res[:, :d].astype(jnp.bfloat16)
    kbf_ref[...] = res[:, d:2 * d].astype(jnp.bfloat16)


def _proj(x, wcat, bcat, d):
    n, din = x.shape
    dout = wcat.shape[1]
    return pl.pallas_call(
        functools.partial(_proj_body, d),
        grid=(n // MB,),
        in_specs=[
            pl.BlockSpec((MB, din), lambda i: (i, 0)),
            pl.BlockSpec((din, dout), lambda i: (0, 0)),
            pl.BlockSpec((1, dout), lambda i: (0, 0)),
        ],
        out_specs=[pl.BlockSpec((MB, dout), lambda i: (i, 0)),
                   pl.BlockSpec((MB, d), lambda i: (i, 0)),
                   pl.BlockSpec((MB, d), lambda i: (i, 0))],
        out_shape=[jax.ShapeDtypeStruct((n, dout), jnp.float32),
                   jax.ShapeDtypeStruct((n, d), jnp.bfloat16),
                   jax.ShapeDtypeStruct((n, d), jnp.bfloat16)],
    )(x, wcat, bcat.reshape(1, dout))


def _edge_body(d, q_hbm, k_hbm, v0_hbm, v1_hbm, v2_hbm, v3_hbm,
               srcr_hbm, dstr_hbm,
               agg_hbm, den_hbm, m_hbm,
               qa, qb, ka, kb, va, vb, lbuf, srcb, dstb, mbuf, mxv,
               zbuf, zdbuf, agg_sh, den_sh, mxs_sh, sema, semb):
    c = lax.axis_index("c")
    s = lax.axis_index("s")
    w = s * 2 + c
    base = w * PW
    inv = 1.0 / math.sqrt(d)
    nsl = d // 16
    nh = d // DV
    lane = jnp.arange(16, dtype=jnp.int32)
    base_row = s * 640

    pltpu.sync_copy(srcr_hbm.at[w], srcb)
    pltpu.sync_copy(dstr_hbm.at[w], dstb)

    def zrow(i, carry):
        zbuf[i % 40, pl.ds((i // 40) * 16, 16)] = jnp.zeros((16,), jnp.float32)
        return carry

    lax.fori_loop(0, 40 * DV // 16, zrow, 0)

    def zden(i, carry):
        zdbuf[pl.ds(i * 16, 16)] = jnp.zeros((16,), jnp.float32)
        return carry

    lax.fori_loop(0, 40, zden, 0)
    for t in range(16):
        pltpu.sync_copy(zbuf, agg_sh.at[pl.ds(base_row + t * 40, 40)])
    pltpu.sync_copy(zdbuf, den_sh.at[pl.ds(s * 640, 640)])

    def _dot_chunk(j, qrow, krow):
        def group(g, carry2):
            lgv = jnp.full((16,), NEG, jnp.float32)
            for p in range(16):
                i = g * 16 + p
                acc = None
                for cc in range(nsl // 2):
                    qv = qrow[i, pl.ds(cc * 32, 32)]
                    kv = krow[i, pl.ds(cc * 32, 32)]
                    q0, q1 = plsc.unpack(qv, format=plsc.PackFormat.INTERLEAVED)
                    k0, k1 = plsc.unpack(kv, format=plsc.PackFormat.INTERLEAVED)
                    t = q0 * k0 + q1 * k1
                    acc = t if acc is None else acc + t
                lg = jnp.sum(acc) * inv
                lgv = jnp.where(lane == p, jnp.full((16,), lg, jnp.float32), lgv)
            eid = base + j * CH + g * 16 + lane
            lgv = jnp.where(eid < E, lgv, jnp.full((16,), NEG, jnp.float32))
            lbuf[pl.ds(j * CH + g * 16, 16)] = lgv
            return carry2

        lax.fori_loop(0, CH // 16, group, 0)

    def _start_qk(j, qrow, krow, sem):
        pltpu.async_copy(q_hbm.at[dstb.at[j]], qrow, sem)
        pltpu.async_copy(k_hbm.at[srcb.at[j]], krow, sem)

    def _wait_qk(qrow, krow, sem):
        pltpu.make_async_copy(q_hbm.at[dstb.at[0]], qrow, sem).wait()
        pltpu.make_async_copy(k_hbm.at[srcb.at[0]], krow, sem).wait()

    _start_qk(0, qa, ka, sema)

    def chunkpair(t, carry):
        j0 = 2 * t
        j1 = 2 * t + 1
        _start_qk(j1, qb, kb, semb)
        _wait_qk(qa, ka, sema)
        _dot_chunk(j0, qa, ka)
        _start_qk(jnp.minimum(j0 + 2, NCHUNK - 1), qa, ka, sema)
        _wait_qk(qb, kb, semb)
        _dot_chunk(j1, qb, kb)
        return carry

    lax.fori_loop(0, NCHUNK // 2, chunkpair, 0)
    _wait_qk(qa, ka, sema)

    def mxstep(i, m):
        return jnp.maximum(m, lbuf[pl.ds(i * 16, 16)])

    wmax = lax.fori_loop(0, PW // 16, mxstep, jnp.full((16,), NEG, jnp.float32))
    mbuf[0, pl.ds(0, 16)] = wmax
    pltpu.sync_copy(mbuf, mxs_sh.at[pl.ds(s, 1)])
    plsc.subcore_barrier()
    pltpu.sync_copy(mxs_sh, mxv)

    def gmx(i, m):
        return jnp.maximum(m, mxv[i, pl.ds(0, 16)])

    gv = lax.fori_loop(0, 16, gmx, jnp.full((16,), NEG, jnp.float32))
    mc = jnp.max(gv)
    mcv = jnp.full((16,), mc, jnp.float32)

    @pl.when(s == 0)
    def _dump_m():
        mbuf[0, pl.ds(0, 16)] = mcv
        pltpu.sync_copy(mbuf, m_hbm.at[c])

    def estep(i, carry):
        lbuf[pl.ds(i * 16, 16)] = jnp.exp(lbuf[pl.ds(i * 16, 16)] - mcv)
        return carry

    lax.fori_loop(0, PW // 16, estep, 0)

    for h in range(nh):
        vh_hbm = (v0_hbm, v1_hbm, v2_hbm, v3_hbm)[h]

        def _scale_scatter(j, vrow, h=h):
            def scale(g, carry2):
                ev = lbuf[pl.ds(j * CH + g * 16, 16)]
                for p in range(16):
                    i = g * 16 + p
                    bc = jnp.full((16,), ev[p], jnp.float32)
                    for cc in range(DV // 16):
                        vrow[i, pl.ds(cc * 16, 16)] = (
                            vrow[i, pl.ds(cc * 16, 16)] * bc)
                return carry2

            lax.fori_loop(0, CH // 16, scale, 0)
            if h == 0:
                pltpu.sync_copy(lbuf.at[pl.ds(j * CH, CH)],
                                den_sh.at[dstb.at[j]], add=True)
            pltpu.sync_copy(vrow, agg_sh.at[dstb.at[j]], add=True)

        def _wait_v(vrow, sem, vh_hbm=vh_hbm):
            pltpu.make_async_copy(vh_hbm.at[srcb.at[0]], vrow, sem).wait()

        pltpu.async_copy(vh_hbm.at[srcb.at[0]], va, sema)

        def chunk2pair(t, carry, vh_hbm=vh_hbm):
            j0 = 2 * t
            j1 = 2 * t + 1
            pltpu.async_copy(vh_hbm.at[srcb.at[j1]], vb, semb)
            _wait_v(va, sema)
            _scale_scatter(j0, va)
            pltpu.async_copy(
                vh_hbm.at[srcb.at[jnp.minimum(j0 + 2, NCHUNK - 1)]], va, sema)
            _wait_v(vb, semb)
            _scale_scatter(j1, vb)
            return carry

        lax.fori_loop(0, NCHUNK // 2, chunk2pair, 0)
        _wait_v(va, sema)
        plsc.subcore_barrier()

        for t in range(5):
            pltpu.sync_copy(agg_sh.at[pl.ds(base_row + t * 128, 128)],
                            agg_hbm.at[c, h, pl.ds(base_row + t * 128, 128)])
        if h == 0:
            pltpu.sync_copy(den_sh.at[pl.ds(s * 640, 640)],
                            den_hbm.at[c, pl.ds(s * 640, 640)])
        if h + 1 < nh:
            for t in range(16):
                pltpu.sync_copy(zbuf, agg_sh.at[pl.ds(base_row + t * 40, 40)])
            plsc.subcore_barrier()


def _make_edge(d):
    mesh = plsc.VectorSubcoreMesh(core_axis_name="c", subcore_axis_name="s")
    return pl.kernel(
        functools.partial(_edge_body, d),
        out_type=[
            jax.ShapeDtypeStruct((2, d // DV, NPAD, DV), jnp.float32),
            jax.ShapeDtypeStruct((2, NPAD), jnp.float32),
            jax.ShapeDtypeStruct((2, 1, 16), jnp.float32),
        ],
        mesh=mesh,
        name=f"edge_d{d}",
        compiler_params=pltpu.CompilerParams(
            needs_layout_passes=False, use_tc_tiling_on_sc=False),
        scratch_types=[
            pltpu.VMEM((CH, d), jnp.bfloat16),
            pltpu.VMEM((CH, d), jnp.bfloat16),
            pltpu.VMEM((CH, d), jnp.bfloat16),
            pltpu.VMEM((CH, d), jnp.bfloat16),
            pltpu.VMEM((CH, DV), jnp.float32),
            pltpu.VMEM((CH, DV), jnp.float32),
            pltpu.VMEM((PW,), jnp.float32),
            pltpu.VMEM((NCHUNK, CH), jnp.int32),
            pltpu.VMEM((NCHUNK, CH), jnp.int32),
            pltpu.VMEM((1, 16), jnp.float32),
            pltpu.VMEM((16, 16), jnp.float32),
            pltpu.VMEM((40, DV), jnp.float32),
            pltpu.VMEM((640,), jnp.float32),
            pltpu.VMEM_SHARED((NPAD, DV), jnp.float32),
            pltpu.VMEM_SHARED((NPAD,), jnp.float32),
            pltpu.VMEM_SHARED((16, 16), jnp.float32),
            pltpu.SemaphoreType.DMA,
            pltpu.SemaphoreType.DMA,
        ],
    )


_EDGE = {d: _make_edge(d) for d in (D1, D2)}


def _edges(qbf, kbf, v, srcr, dstr, d):
    vs = [v[:, h * DV:(h + 1) * DV] for h in range(d // DV)]
    while len(vs) < 4:
        vs.append(vs[0])
    agg_p, den_p, m_p = _EDGE[d](qbf, kbf, *vs, srcr, dstr)
    return agg_p, den_p.reshape(2, NPAD, 1), m_p


def _rescale(m_ref):
    mv = m_ref[...]
    mm = jnp.max(mv)
    sv = jnp.exp(mv - mm)
    return sv[0, 0, 0], sv[1, 0, 0]


def _glue_body(agg_ref, denp_ref, m_ref, s_ref, w_ref, b_ref, o_ref,
               qbf_ref, kbf_ref):
    s0, s1 = _rescale(m_ref)
    agg = (jnp.concatenate([agg_ref[0, h] for h in range(4)], axis=1) * s0
           + jnp.concatenate([agg_ref[1, h] for h in range(4)], axis=1) * s1)
    den = denp_ref[0] * s0 + denp_ref[1] * s1
    h1 = _elu(agg / (den + 1e-16) + s_ref[...])
    res = (
        jnp.dot(h1, w_ref[...], preferred_element_type=jnp.float32) + b_ref[...]
    )
    o_ref[...] = res
    qbf_ref[...] = res[:, :D2].astype(jnp.bfloat16)
    kbf_ref[...] = res[:, D2:2 * D2].astype(jnp.bfloat16)


def _glue(agg_p, den_p, m_p, skip, wcat, bcat):
    n, din = skip.shape
    dout = wcat.shape[1]
    return pl.pallas_call(
        _glue_body,
        grid=(n // MB,),
        in_specs=[
            pl.BlockSpec((2, 4, MB, DV), lambda i: (0, 0, i, 0)),
            pl.BlockSpec((2, MB, 1), lambda i: (0, i, 0)),
            pl.BlockSpec((2, 1, 16), lambda i: (0, 0, 0)),
            pl.BlockSpec((MB, din), lambda i: (i, 0)),
            pl.BlockSpec((din, dout), lambda i: (0, 0)),
            pl.BlockSpec((1, dout), lambda i: (0, 0)),
        ],
        out_specs=[pl.BlockSpec((MB, dout), lambda i: (i, 0)),
                   pl.BlockSpec((MB, D2), lambda i: (i, 0)),
                   pl.BlockSpec((MB, D2), lambda i: (i, 0))],
        out_shape=[jax.ShapeDtypeStruct((n, dout), jnp.float32),
                   jax.ShapeDtypeStruct((n, D2), jnp.bfloat16),
                   jax.ShapeDtypeStruct((n, D2), jnp.bfloat16)],
    )(agg_p, den_p, m_p, skip, wcat, bcat.reshape(1, dout))


def _mlp3(h, w1, b1, w2, b2, w3, b3):
    h = jax.nn.relu(jnp.dot(h, w1, preferred_element_type=jnp.float32) + b1)
    h = jax.nn.relu(jnp.dot(h, w2, preferred_element_type=jnp.float32) + b2)
    return jnp.dot(h, w3, preferred_element_type=jnp.float32) + b3


def _head_body(agg_ref, denp_ref, m_ref, s_ref, wb_ref, bb_ref, wfc_ref,
               bfc_ref, wd1_ref, bd1_ref, wd2_ref, bd2_ref, wd3_ref, bd3_ref,
               wc1_ref, bc1_ref, wc2_ref, bc2_ref, wc3_ref, bc3_ref,
               emb_ref, dom_ref, cls0_ref, cls1_ref):
    s0, s1 = _rescale(m_ref)
    agg = (jnp.concatenate([agg_ref[0, 0], agg_ref[0, 1]], axis=1) * s0
           + jnp.concatenate([agg_ref[1, 0], agg_ref[1, 1]], axis=1) * s1)
    den = denp_ref[0] * s0 + denp_ref[1] * s1
    h2 = _elu(agg / (den + 1e-16) + s_ref[...])
    ss = jnp.dot(h2, wb_ref[...], preferred_element_type=jnp.float32) + bb_ref[...]
    emb = jnp.dot(ss, wfc_ref[...], preferred_element_type=jnp.float32) + bfc_ref[...]
    emb_ref[...] = emb
    m = jnp.max(emb, axis=1, keepdims=True)
    ex = jnp.exp(emb - m)
    p = ex / jnp.sum(ex, axis=1, keepdims=True)
    dom_ref[...] = _mlp3(ss, wd1_ref[...], bd1_ref[...], wd2_ref[...],
                         bd2_ref[...], wd3_ref[...], bd3_ref[...])
    for i, cls_ref in ((0, cls0_ref), (1, cls1_ref)):
        f = p[:, i][:, None] * ss
        cls_ref[...] = _mlp3(f, wc1_ref[i], bc1_ref[i], wc2_ref[i],
                             bc2_ref[i], wc3_ref[i], bc3_ref[i])


def _heads(agg_p, den_p, m_p, s2, Wb, bb, Wfc, bfc, Wd1, bd1, Wd2, bd2,
           Wd3, bd3, Wc1, bc1, Wc2, bc2, Wc3, bc3):
    n = s2.shape[0]
    full = lambda *shape: pl.BlockSpec(shape, lambda i: (0,) * len(shape))
    row = lambda d: pl.BlockSpec((MB, d), lambda i: (i, 0))
    out_shapes = [jax.ShapeDtypeStruct((n, 2), jnp.float32)] * 4
    return pl.pallas_call(
        _head_body,
        grid=(n // MB,),
        in_specs=[
            pl.BlockSpec((2, 2, MB, DV), lambda i: (0, 0, i, 0)),
            pl.BlockSpec((2, MB, 1), lambda i: (0, i, 0)),
            pl.BlockSpec((2, 1, 16), lambda i: (0, 0, 0)),
            row(D2),
            full(D2, 32), full(1, 32), full(32, NC), full(1, NC),
            full(32, 512), full(1, 512), full(512, 512), full(1, 512),
            full(512, 2), full(1, 2),
            full(NC, 32, 512), full(NC, 1, 512),
            full(NC, 512, 512), full(NC, 1, 512),
            full(NC, 512, 2), full(NC, 1, 2),
        ],
        out_specs=[row(2), row(2), row(2), row(2)],
        out_shape=out_shapes,
    )(agg_p, den_p, m_p, s2, Wb, bb.reshape(1, 32), Wfc, bfc.reshape(1, NC),
      Wd1, bd1.reshape(1, 512), Wd2, bd2.reshape(1, 512),
      Wd3, bd3.reshape(1, 2),
      Wc1, bc1.reshape(NC, 1, 512), Wc2, bc2.reshape(NC, 1, 512),
      Wc3, bc3.reshape(NC, 1, 2))


def _pad_edges(ei):
    pad = jnp.arange(E_PAD - E, dtype=jnp.int32) % N
    src = jnp.concatenate([ei[0], pad]).reshape(NW, NCHUNK, CH)
    dst = jnp.concatenate([ei[1], pad]).reshape(NW, NCHUNK, CH)
    return src, dst


def kernel(source_x, target_x, source_edge_index, target_edge_index,
           Wq1, bq1, Wk1, bk1, Wv1, bv1, Ws1, bs1,
           Wq2, bq2, Wk2, bk2, Wv2, bv2, Ws2, bs2,
           Wb, bb, Wfc, bfc,
           Wd1, bd1, Wd2, bd2, Wd3, bd3,
           Wc1, bc1, Wc2, bc2, Wc3, bc3):
    wcat1 = jnp.concatenate([Wq1, Wk1, Wv1, Ws1], axis=1)
    bcat1 = jnp.concatenate([bq1, bk1, bv1, bs1])
    wcat2 = jnp.concatenate([Wq2, Wk2, Wv2, Ws2], axis=1)
    bcat2 = jnp.concatenate([bq2, bk2, bv2, bs2])

    s_srcr, s_dstr = _pad_edges(source_edge_index)
    t_srcr, t_dstr = _pad_edges(target_edge_index)
    s_qkvs1, s_qbf1, s_kbf1 = _proj(source_x, wcat1, bcat1, D1)
    t_qkvs1, t_qbf1, t_kbf1 = _proj(target_x, wcat1, bcat1, D1)
    s_agg1, s_den1, s_m1 = _edges(s_qbf1, s_kbf1,
                                  s_qkvs1[:, 2 * D1:3 * D1],
                                  s_srcr, s_dstr, D1)
    t_agg1, t_den1, t_m1 = _edges(t_qbf1, t_kbf1,
                                  t_qkvs1[:, 2 * D1:3 * D1],
                                  t_srcr, t_dstr, D1)
    s_qkvs2, s_qbf2, s_kbf2 = _glue(s_agg1, s_den1, s_m1,
                                    s_qkvs1[:, 3 * D1:], wcat2, bcat2)
    s_agg2, s_den2, s_m2 = _edges(s_qbf2, s_kbf2,
                                  s_qkvs2[:, 2 * D2:3 * D2],
                                  s_srcr, s_dstr, D2)
    t_qkvs2, t_qbf2, t_kbf2 = _glue(t_agg1, t_den1, t_m1,
                                    t_qkvs1[:, 3 * D1:], wcat2, bcat2)
    t_agg2, t_den2, t_m2 = _edges(t_qbf2, t_kbf2,
                                  t_qkvs2[:, 2 * D2:3 * D2],
                                  t_srcr, t_dstr, D2)
    s_s2 = s_qkvs2[:, 3 * D2:]
    t_s2 = t_qkvs2[:, 3 * D2:]

    head_w = (Wb, bb, Wfc, bfc, Wd1, bd1, Wd2, bd2, Wd3, bd3,
              Wc1, bc1, Wc2, bc2, Wc3, bc3)
    s_emb, s_dom, s_cls0, s_cls1 = _heads(s_agg2, s_den2, s_m2, s_s2, *head_w)
    _, t_dom, t_cls0, t_cls1 = _heads(t_agg2, t_den2, t_m2, t_s2, *head_w)

    s_out = jnp.stack([s_cls0, s_cls1])
    t_out = jnp.stack([t_cls0, t_cls1])
    return (s_emb, s_dom, t_dom, s_out, t_out)


# speedup vs baseline: 1.1441x; 1.1215x over previous
import functools
import math

import jax
import jax.numpy as jnp
from jax import lax
from jax.experimental import pallas as pl
from jax.experimental.pallas import tpu as pltpu
from jax.experimental.pallas import tpu_sc as plsc

N = 10000
E = 160000
IN_DIM = 128
D1 = 128
D2 = 64
NC = 2
MB = 1000

NW = 32
E_PAD = 163840
PW = E_PAD // NW
CH = 256
NCHUNK = PW // CH
NPAD = 10240
DV = 32
NEG = -1e30


def _elu(x):
    return jnp.where(x > 0, x, jnp.exp(x) - 1.0)


def _proj_body(d, x_ref, w_ref, b_ref, o_ref, qbf_ref, kbf_ref):
    res = (
        jnp.dot(x_ref[...], w_ref[...], preferred_element_type=jnp.float32)
        + b_ref[...]
    )
    o_ref[...] = res
    qbf_ref[...] = res[:, :d].astype(jnp.bfloat16)
    kbf_ref[...] = res[:, d:2 * d].astype(jnp.bfloat16)


def _proj(x, wcat, bcat, d):
    n, din = x.shape
    dout = wcat.shape[1]
    return pl.pallas_call(
        functools.partial(_proj_body, d),
        grid=(n // MB,),
        in_specs=[
            pl.BlockSpec((MB, din), lambda i: (i, 0)),
            pl.BlockSpec((din, dout), lambda i: (0, 0)),
            pl.BlockSpec((1, dout), lambda i: (0, 0)),
        ],
        out_specs=[pl.BlockSpec((MB, dout), lambda i: (i, 0)),
                   pl.BlockSpec((MB, d), lambda i: (i, 0)),
                   pl.BlockSpec((MB, d), lambda i: (i, 0))],
        out_shape=[jax.ShapeDtypeStruct((n, dout), jnp.float32),
                   jax.ShapeDtypeStruct((n, d), jnp.bfloat16),
                   jax.ShapeDtypeStruct((n, d), jnp.bfloat16)],
    )(x, wcat, bcat.reshape(1, dout))


def _edge_body(d, q_hbm, k_hbm, v0_hbm, v1_hbm, v2_hbm, v3_hbm,
               srcr_hbm, dstr_hbm,
               agg_hbm, den_hbm, m_hbm,
               qa, qb, ka, kb, va, vb, lbuf, srcb, dstb, mbuf, mxv,
               zbuf, zdbuf, agg_sh, den_sh, mxs_sh, sema, semb):
    c = lax.axis_index("c")
    s = lax.axis_index("s")
    w = s * 2 + c
    base = w * PW
    inv = 1.0 / math.sqrt(d)
    nsl = d // 16
    nh = d // DV
    lane = jnp.arange(16, dtype=jnp.int32)
    base_row = s * 640

    pltpu.sync_copy(srcr_hbm.at[w], srcb)
    pltpu.sync_copy(dstr_hbm.at[w], dstb)

    def zrow(i, carry):
        zbuf[i % 40, pl.ds((i // 40) * 16, 16)] = jnp.zeros((16,), jnp.float32)
        return carry

    lax.fori_loop(0, 40 * DV // 16, zrow, 0)

    def zden(i, carry):
        zdbuf[pl.ds(i * 16, 16)] = jnp.zeros((16,), jnp.float32)
        return carry

    lax.fori_loop(0, 40, zden, 0)
    for t in range(16):
        pltpu.sync_copy(zbuf, agg_sh.at[pl.ds(base_row + t * 40, 40)])
    pltpu.sync_copy(zdbuf, den_sh.at[pl.ds(s * 640, 640)])

    def _dot_chunk(j, qrow, krow):
        def group(g, carry2):
            lgv = jnp.full((16,), NEG, jnp.float32)
            for p in range(16):
                i = g * 16 + p
                acc = None
                for cc in range(nsl // 2):
                    qv = qrow[i, pl.ds(cc * 32, 32)]
                    kv = krow[i, pl.ds(cc * 32, 32)]
                    q0, q1 = plsc.unpack(qv, format=plsc.PackFormat.INTERLEAVED)
                    k0, k1 = plsc.unpack(kv, format=plsc.PackFormat.INTERLEAVED)
                    t = q0 * k0 + q1 * k1
                    acc = t if acc is None else acc + t
                lg = jnp.sum(acc) * inv
                lgv = jnp.where(lane == p, jnp.full((16,), lg, jnp.float32), lgv)
            eid = base + j * CH + g * 16 + lane
            lgv = jnp.where(eid < E, lgv, jnp.full((16,), NEG, jnp.float32))
            lbuf[pl.ds(j * CH + g * 16, 16)] = lgv
            return carry2

        lax.fori_loop(0, CH // 16, group, 0)

    def _start_qk(j, qrow, krow, sem):
        pltpu.async_copy(q_hbm.at[dstb.at[j]], qrow, sem)
        pltpu.async_copy(k_hbm.at[srcb.at[j]], krow, sem)

    def _wait_qk(qrow, krow, sem):
        pltpu.make_async_copy(q_hbm.at[dstb.at[0]], qrow, sem).wait()
        pltpu.make_async_copy(k_hbm.at[srcb.at[0]], krow, sem).wait()

    _start_qk(0, qa, ka, sema)

    def chunkpair(t, carry):
        j0 = 2 * t
        j1 = 2 * t + 1
        _start_qk(j1, qb, kb, semb)
        _wait_qk(qa, ka, sema)
        _dot_chunk(j0, qa, ka)
        _start_qk(jnp.minimum(j0 + 2, NCHUNK - 1), qa, ka, sema)
        _wait_qk(qb, kb, semb)
        _dot_chunk(j1, qb, kb)
        return carry

    lax.fori_loop(0, NCHUNK // 2, chunkpair, 0)
    _wait_qk(qa, ka, sema)

    def mxstep(i, m):
        return jnp.maximum(m, lbuf[pl.ds(i * 16, 16)])

    wmax = lax.fori_loop(0, PW // 16, mxstep, jnp.full((16,), NEG, jnp.float32))
    mbuf[0, pl.ds(0, 16)] = wmax
    pltpu.sync_copy(mbuf, mxs_sh.at[pl.ds(s, 1)])
    plsc.subcore_barrier()
    pltpu.sync_copy(mxs_sh, mxv)

    def gmx(i, m):
        return jnp.maximum(m, mxv[i, pl.ds(0, 16)])

    gv = lax.fori_loop(0, 16, gmx, jnp.full((16,), NEG, jnp.float32))
    mc = jnp.max(gv)
    mcv = jnp.full((16,), mc, jnp.float32)

    @pl.when(s == 0)
    def _dump_m():
        mbuf[0, pl.ds(0, 16)] = mcv
        pltpu.sync_copy(mbuf, m_hbm.at[c])

    def estep(i, carry):
        lbuf[pl.ds(i * 16, 16)] = jnp.exp(lbuf[pl.ds(i * 16, 16)] - mcv)
        return carry

    lax.fori_loop(0, PW // 16, estep, 0)

    for h in range(nh):
        vh_hbm = (v0_hbm, v1_hbm, v2_hbm, v3_hbm)[h]

        def _scale_scatter(j, vrow, h=h):
            def scale(g, carry2):
                ev = lbuf[pl.ds(j * CH + g * 16, 16)]
                for p in range(16):
                    i = g * 16 + p
                    bc = jnp.full((16,), ev[p], jnp.float32)
                    for cc in range(DV // 16):
                        vrow[i, pl.ds(cc * 16, 16)] = (
                            vrow[i, pl.ds(cc * 16, 16)] * bc)
                return carry2

            lax.fori_loop(0, CH // 16, scale, 0)
            if h == 0:
                pltpu.sync_copy(lbuf.at[pl.ds(j * CH, CH)],
                                den_sh.at[dstb.at[j]], add=True)
            pltpu.sync_copy(vrow, agg_sh.at[dstb.at[j]], add=True)

        def _wait_v(vrow, sem, vh_hbm=vh_hbm):
            pltpu.make_async_copy(vh_hbm.at[srcb.at[0]], vrow, sem).wait()

        pltpu.async_copy(vh_hbm.at[srcb.at[0]], va, sema)

        def chunk2pair(t, carry, vh_hbm=vh_hbm):
            j0 = 2 * t
            j1 = 2 * t + 1
            pltpu.async_copy(vh_hbm.at[srcb.at[j1]], vb, semb)
            _wait_v(va, sema)
            _scale_scatter(j0, va)
            pltpu.async_copy(
                vh_hbm.at[srcb.at[jnp.minimum(j0 + 2, NCHUNK - 1)]], va, sema)
            _wait_v(vb, semb)
            _scale_scatter(j1, vb)
            return carry

        lax.fori_loop(0, NCHUNK // 2, chunk2pair, 0)
        _wait_v(va, sema)
        plsc.subcore_barrier()

        for t in range(5):
            pltpu.sync_copy(agg_sh.at[pl.ds(base_row + t * 128, 128)],
                            agg_hbm.at[c, h, pl.ds(base_row + t * 128, 128)])
        if h == 0:
            pltpu.sync_copy(den_sh.at[pl.ds(s * 640, 640)],
                            den_hbm.at[c, pl.ds(s * 640, 640)])
        if h + 1 < nh:
            for t in range(16):
                pltpu.sync_copy(zbuf, agg_sh.at[pl.ds(base_row + t * 40, 40)])
            plsc.subcore_barrier()


def _make_edge(d):
    mesh = plsc.VectorSubcoreMesh(core_axis_name="c", subcore_axis_name="s")
    return pl.kernel(
        functools.partial(_edge_body, d),
        out_type=[
            jax.ShapeDtypeStruct((2, d // DV, NPAD, DV), jnp.float32),
            jax.ShapeDtypeStruct((2, NPAD), jnp.float32),
            jax.ShapeDtypeStruct((2, 1, 16), jnp.float32),
        ],
        mesh=mesh,
        name=f"edge_d{d}",
        compiler_params=pltpu.CompilerParams(
            needs_layout_passes=False, use_tc_tiling_on_sc=False),
        scratch_types=[
            pltpu.VMEM((CH, d), jnp.bfloat16),
            pltpu.VMEM((CH, d), jnp.bfloat16),
            pltpu.VMEM((CH, d), jnp.bfloat16),
            pltpu.VMEM((CH, d), jnp.bfloat16),
            pltpu.VMEM((CH, DV), jnp.float32),
            pltpu.VMEM((CH, DV), jnp.float32),
            pltpu.VMEM((PW,), jnp.float32),
            pltpu.VMEM((NCHUNK, CH), jnp.int32),
            pltpu.VMEM((NCHUNK, CH), jnp.int32),
            pltpu.VMEM((1, 16), jnp.float32),
            pltpu.VMEM((16, 16), jnp.float32),
            pltpu.VMEM((40, DV), jnp.float32),
            pltpu.VMEM((640,), jnp.float32),
            pltpu.VMEM_SHARED((NPAD, DV), jnp.float32),
            pltpu.VMEM_SHARED((NPAD,), jnp.float32),
            pltpu.VMEM_SHARED((16, 16), jnp.float32),
            pltpu.SemaphoreType.DMA,
            pltpu.SemaphoreType.DMA,
        ],
    )


_EDGE = {d: _make_edge(d) for d in (D1, D2)}


def _edges(qbf, kbf, v, srcr, dstr, d):
    vs = [v[:, h * DV:(h + 1) * DV] for h in range(d // DV)]
    while len(vs) < 4:
        vs.append(vs[0])
    agg_p, den_p, m_p = _EDGE[d](qbf, kbf, *vs, srcr, dstr)
    return agg_p, den_p.reshape(2, NPAD, 1), m_p


def _rescale(m_ref):
    mv = m_ref[...]
    mm = jnp.max(mv)
    sv = jnp.exp(mv - mm)
    return sv[0, 0, 0], sv[1, 0, 0]


def _glue_body(agg_ref, denp_ref, m_ref, s_ref, w_ref, b_ref, o_ref,
               qbf_ref, kbf_ref):
    s0, s1 = _rescale(m_ref)
    agg = (jnp.concatenate([agg_ref[0, h] for h in range(4)], axis=1) * s0
           + jnp.concatenate([agg_ref[1, h] for h in range(4)], axis=1) * s1)
    den = denp_ref[0] * s0 + denp_ref[1] * s1
    h1 = _elu(agg / (den + 1e-16) + s_ref[...])
    res = (
        jnp.dot(h1, w_ref[...], preferred_element_type=jnp.float32) + b_ref[...]
    )
    o_ref[...] = res
    qbf_ref[...] = res[:, :D2].astype(jnp.bfloat16)
    kbf_ref[...] = res[:, D2:2 * D2].astype(jnp.bfloat16)


def _glue(agg_p, den_p, m_p, skip, wcat, bcat):
    n, din = skip.shape
    dout = wcat.shape[1]
    return pl.pallas_call(
        _glue_body,
        grid=(n // MB,),
        in_specs=[
            pl.BlockSpec((2, 4, MB, DV), lambda i: (0, 0, i, 0)),
            pl.BlockSpec((2, MB, 1), lambda i: (0, i, 0)),
            pl.BlockSpec((2, 1, 16), lambda i: (0, 0, 0)),
            pl.BlockSpec((MB, din), lambda i: (i, 0)),
            pl.BlockSpec((din, dout), lambda i: (0, 0)),
            pl.BlockSpec((1, dout), lambda i: (0, 0)),
        ],
        out_specs=[pl.BlockSpec((MB, dout), lambda i: (i, 0)),
                   pl.BlockSpec((MB, D2), lambda i: (i, 0)),
                   pl.BlockSpec((MB, D2), lambda i: (i, 0))],
        out_shape=[jax.ShapeDtypeStruct((n, dout), jnp.float32),
                   jax.ShapeDtypeStruct((n, D2), jnp.bfloat16),
                   jax.ShapeDtypeStruct((n, D2), jnp.bfloat16)],
    )(agg_p, den_p, m_p, skip, wcat, bcat.reshape(1, dout))


def _mlp3(h, w1, b1, w2, b2, w3, b3):
    h = jax.nn.relu(jnp.dot(h, w1, preferred_element_type=jnp.float32) + b1)
    h = jax.nn.relu(jnp.dot(h, w2, preferred_element_type=jnp.float32) + b2)
    return jnp.dot(h, w3, preferred_element_type=jnp.float32) + b3


def _head_body(agg_ref, denp_ref, m_ref, s_ref, wb_ref, bb_ref, wfc_ref,
               bfc_ref, wd1_ref, bd1_ref, wd2_ref, bd2_ref, wd3_ref, bd3_ref,
               wc1_ref, bc1_ref, wc2_ref, bc2_ref, wc3_ref, bc3_ref,
               emb_ref, dom_ref, cls0_ref, cls1_ref):
    s0, s1 = _rescale(m_ref)
    agg = (jnp.concatenate([agg_ref[0, 0], agg_ref[0, 1]], axis=1) * s0
           + jnp.concatenate([agg_ref[1, 0], agg_ref[1, 1]], axis=1) * s1)
    den = denp_ref[0] * s0 + denp_ref[1] * s1
    h2 = _elu(agg / (den + 1e-16) + s_ref[...])
    ss = jnp.dot(h2, wb_ref[...], preferred_element_type=jnp.float32) + bb_ref[...]
    emb = jnp.dot(ss, wfc_ref[...], preferred_element_type=jnp.float32) + bfc_ref[...]
    emb_ref[...] = emb
    m = jnp.max(emb, axis=1, keepdims=True)
    ex = jnp.exp(emb - m)
    p = ex / jnp.sum(ex, axis=1, keepdims=True)
    dom_ref[...] = _mlp3(ss, wd1_ref[...], bd1_ref[...], wd2_ref[...],
                         bd2_ref[...], wd3_ref[...], bd3_ref[...])
    for i, cls_ref in ((0, cls0_ref), (1, cls1_ref)):
        f = p[:, i][:, None] * ss
        cls_ref[...] = _mlp3(f, wc1_ref[i], bc1_ref[i], wc2_ref[i],
                             bc2_ref[i], wc3_ref[i], bc3_ref[i])


def _heads(agg_p, den_p, m_p, s2, Wb, bb, Wfc, bfc, Wd1, bd1, Wd2, bd2,
           Wd3, bd3, Wc1, bc1, Wc2, bc2, Wc3, bc3):
    n = s2.shape[0]
    full = lambda *shape: pl.BlockSpec(shape, lambda i: (0,) * len(shape))
    row = lambda d: pl.BlockSpec((MB, d), lambda i: (i, 0))
    out_shapes = [jax.ShapeDtypeStruct((n, 2), jnp.float32)] * 4
    return pl.pallas_call(
        _head_body,
        grid=(n // MB,),
        in_specs=[
            pl.BlockSpec((2, 2, MB, DV), lambda i: (0, 0, i, 0)),
            pl.BlockSpec((2, MB, 1), lambda i: (0, i, 0)),
            pl.BlockSpec((2, 1, 16), lambda i: (0, 0, 0)),
            row(D2),
            full(D2, 32), full(1, 32), full(32, NC), full(1, NC),
            full(32, 512), full(1, 512), full(512, 512), full(1, 512),
            full(512, 2), full(1, 2),
            full(NC, 32, 512), full(NC, 1, 512),
            full(NC, 512, 512), full(NC, 1, 512),
            full(NC, 512, 2), full(NC, 1, 2),
        ],
        out_specs=[row(2), row(2), row(2), row(2)],
        out_shape=out_shapes,
    )(agg_p, den_p, m_p, s2, Wb, bb.reshape(1, 32), Wfc, bfc.reshape(1, NC),
      Wd1, bd1.reshape(1, 512), Wd2, bd2.reshape(1, 512),
      Wd3, bd3.reshape(1, 2),
      Wc1, bc1.reshape(NC, 1, 512), Wc2, bc2.reshape(NC, 1, 512),
      Wc3, bc3.reshape(NC, 1, 2))


def _pad_edges(ei):
    pad = jnp.arange(E_PAD - E, dtype=jnp.int32) % N
    src = jnp.concatenate([ei[0], pad]).reshape(NW, NCHUNK, CH)
    dst = jnp.concatenate([ei[1], pad]).reshape(NW, NCHUNK, CH)
    return src, dst


def kernel(source_x, target_x, source_edge_index, target_edge_index,
           Wq1, bq1, Wk1, bk1, Wv1, bv1, Ws1, bs1,
           Wq2, bq2, Wk2, bk2, Wv2, bv2, Ws2, bs2,
           Wb, bb, Wfc, bfc,
           Wd1, bd1, Wd2, bd2, Wd3, bd3,
           Wc1, bc1, Wc2, bc2, Wc3, bc3):
    wcat1 = jnp.concatenate([Wq1, Wk1, Wv1, Ws1], axis=1)
    bcat1 = jnp.concatenate([bq1, bk1, bv1, bs1])
    wcat2 = jnp.concatenate([Wq2, Wk2, Wv2, Ws2], axis=1)
    bcat2 = jnp.concatenate([bq2, bk2, bv2, bs2])

    s_srcr, s_dstr = _pad_edges(source_edge_index)
    t_srcr, t_dstr = _pad_edges(target_edge_index)
    s_qkvs1, s_qbf1, s_kbf1 = _proj(source_x, wcat1, bcat1, D1)
    t_qkvs1, t_qbf1, t_kbf1 = _proj(target_x, wcat1, bcat1, D1)
    s_agg1, s_den1, s_m1 = _edges(s_qbf1, s_kbf1,
                                  s_qkvs1[:, 2 * D1:3 * D1],
                                  s_srcr, s_dstr, D1)
    t_agg1, t_den1, t_m1 = _edges(t_qbf1, t_kbf1,
                                  t_qkvs1[:, 2 * D1:3 * D1],
                                  t_srcr, t_dstr, D1)
    s_qkvs2, s_qbf2, s_kbf2 = _glue(s_agg1, s_den1, s_m1,
                                    s_qkvs1[:, 3 * D1:], wcat2, bcat2)
    s_agg2, s_den2, s_m2 = _edges(s_qbf2, s_kbf2,
                                  s_qkvs2[:, 2 * D2:3 * D2],
                                  s_srcr, s_dstr, D2)
    t_qkvs2, t_qbf2, t_kbf2 = _glue(t_agg1, t_den1, t_m1,
                                    t_qkvs1[:, 3 * D1:], wcat2, bcat2)
    t_agg2, t_den2, t_m2 = _edges(t_qbf2, t_kbf2,
                                  t_qkvs2[:, 2 * D2:3 * D2],
                                  t_srcr, t_dstr, D2)
    s_s2 = s_qkvs2[:, 3 * D2:]
    t_s2 = t_qkvs2[:, 3 * D2:]

    head_w = (Wb, bb, Wfc, bfc, Wd1, bd1, Wd2, bd2, Wd3, bd3,
              Wc1, bc1, Wc2, bc2, Wc3, bc3)
    s_emb, s_dom, s_cls0, s_cls1 = _heads(s_agg2, s_den2, s_m2, s_s2, *head_w)
    _, t_dom, t_cls0, t_cls1 = _heads(t_agg2, t_den2, t_m2, t_s2, *head_w)

    s_out = jnp.stack([s_cls0, s_cls1])
    t_out = jnp.stack([t_cls0, t_cls1])
    return (s_emb, s_dom, t_dom, s_out, t_out)


# speedup vs baseline: 1.1508x; 1.0059x over previous
import functools
import math

import jax
import jax.numpy as jnp
from jax import lax
from jax.experimental import pallas as pl
from jax.experimental.pallas import tpu as pltpu
from jax.experimental.pallas import tpu_sc as plsc

N = 10000
E = 160000
IN_DIM = 128
D1 = 128
D2 = 64
NC = 2
MB = 1000

NW = 32
E_PAD = 163840
PW = E_PAD // NW
CH_D = {128: 256, 64: 320}
NPAD = 10240
DV = 32
NEG = -1e30


def _elu(x):
    return jnp.where(x > 0, x, jnp.exp(x) - 1.0)


def _proj_body(d, x_ref, w_ref, b_ref, o_ref, qbf_ref, kbf_ref):
    res = (
        jnp.dot(x_ref[...], w_ref[...], preferred_element_type=jnp.float32)
        + b_ref[...]
    )
    o_ref[...] = res
    qbf_ref[...] = res[:, :d].astype(jnp.bfloat16)
    kbf_ref[...] = res[:, d:2 * d].astype(jnp.bfloat16)


def _proj(x, wcat, bcat, d):
    n, din = x.shape
    dout = wcat.shape[1]
    return pl.pallas_call(
        functools.partial(_proj_body, d),
        grid=(n // MB,),
        in_specs=[
            pl.BlockSpec((MB, din), lambda i: (i, 0)),
            pl.BlockSpec((din, dout), lambda i: (0, 0)),
            pl.BlockSpec((1, dout), lambda i: (0, 0)),
        ],
        out_specs=[pl.BlockSpec((MB, dout), lambda i: (i, 0)),
                   pl.BlockSpec((MB, d), lambda i: (i, 0)),
                   pl.BlockSpec((MB, d), lambda i: (i, 0))],
        out_shape=[jax.ShapeDtypeStruct((n, dout), jnp.float32),
                   jax.ShapeDtypeStruct((n, d), jnp.bfloat16),
                   jax.ShapeDtypeStruct((n, d), jnp.bfloat16)],
    )(x, wcat, bcat.reshape(1, dout))


def _edge_body(d, q_hbm, k_hbm, v0_hbm, v1_hbm, v2_hbm, v3_hbm,
               srcr_hbm, dstr_hbm,
               agg_hbm, den_hbm, m_hbm,
               qa, qb, ka, kb, va, vb, lbuf, srcb, dstb, mbuf, mxv,
               zbuf, zdbuf, agg_sh, den_sh, mxs_sh, sema, semb):
    c = lax.axis_index("c")
    s = lax.axis_index("s")
    w = s * 2 + c
    base = w * PW
    inv = 1.0 / math.sqrt(d)
    CH = CH_D[d]
    NCHUNK = PW // CH
    nsl = d // 16
    nh = d // DV
    lane = jnp.arange(16, dtype=jnp.int32)
    base_row = s * 640

    pltpu.sync_copy(srcr_hbm.at[w], srcb)
    pltpu.sync_copy(dstr_hbm.at[w], dstb)

    def zrow(i, carry):
        zbuf[i % 40, pl.ds((i // 40) * 16, 16)] = jnp.zeros((16,), jnp.float32)
        return carry

    lax.fori_loop(0, 40 * DV // 16, zrow, 0)

    def zden(i, carry):
        zdbuf[pl.ds(i * 16, 16)] = jnp.zeros((16,), jnp.float32)
        return carry

    lax.fori_loop(0, 40, zden, 0)
    for t in range(16):
        pltpu.sync_copy(zbuf, agg_sh.at[pl.ds(base_row + t * 40, 40)])
    pltpu.sync_copy(zdbuf, den_sh.at[pl.ds(s * 640, 640)])

    def _dot_chunk(j, qrow, krow):
        def group(g, carry2):
            lgv = jnp.full((16,), NEG, jnp.float32)
            for p in range(16):
                i = g * 16 + p
                acc = None
                for cc in range(nsl // 2):
                    qv = qrow[i, pl.ds(cc * 32, 32)]
                    kv = krow[i, pl.ds(cc * 32, 32)]
                    q0, q1 = plsc.unpack(qv, format=plsc.PackFormat.INTERLEAVED)
                    k0, k1 = plsc.unpack(kv, format=plsc.PackFormat.INTERLEAVED)
                    t = q0 * k0 + q1 * k1
                    acc = t if acc is None else acc + t
                lg = jnp.sum(acc) * inv
                lgv = jnp.where(lane == p, jnp.full((16,), lg, jnp.float32), lgv)
            eid = base + j * CH + g * 16 + lane
            lgv = jnp.where(eid < E, lgv, jnp.full((16,), NEG, jnp.float32))
            lbuf[pl.ds(j * CH + g * 16, 16)] = lgv
            return carry2

        lax.fori_loop(0, CH // 16, group, 0)

    def _start_qk(j, qrow, krow, sem):
        pltpu.async_copy(q_hbm.at[dstb.at[j]], qrow, sem)
        pltpu.async_copy(k_hbm.at[srcb.at[j]], krow, sem)

    def _wait_qk(qrow, krow, sem):
        pltpu.make_async_copy(q_hbm.at[dstb.at[0]], qrow, sem).wait()
        pltpu.make_async_copy(k_hbm.at[srcb.at[0]], krow, sem).wait()

    _start_qk(0, qa, ka, sema)

    def chunkpair(t, carry):
        j0 = 2 * t
        j1 = 2 * t + 1
        _start_qk(j1, qb, kb, semb)
        _wait_qk(qa, ka, sema)
        _dot_chunk(j0, qa, ka)
        _start_qk(jnp.minimum(j0 + 2, NCHUNK - 1), qa, ka, sema)
        _wait_qk(qb, kb, semb)
        _dot_chunk(j1, qb, kb)
        return carry

    lax.fori_loop(0, NCHUNK // 2, chunkpair, 0)
    _wait_qk(qa, ka, sema)

    def mxstep(i, m):
        return jnp.maximum(m, lbuf[pl.ds(i * 16, 16)])

    wmax = lax.fori_loop(0, PW // 16, mxstep, jnp.full((16,), NEG, jnp.float32))
    mbuf[0, pl.ds(0, 16)] = wmax
    pltpu.sync_copy(mbuf, mxs_sh.at[pl.ds(s, 1)])
    plsc.subcore_barrier()
    pltpu.sync_copy(mxs_sh, mxv)

    def gmx(i, m):
        return jnp.maximum(m, mxv[i, pl.ds(0, 16)])

    gv = lax.fori_loop(0, 16, gmx, jnp.full((16,), NEG, jnp.float32))
    mc = jnp.max(gv)
    mcv = jnp.full((16,), mc, jnp.float32)

    @pl.when(s == 0)
    def _dump_m():
        mbuf[0, pl.ds(0, 16)] = mcv
        pltpu.sync_copy(mbuf, m_hbm.at[c])

    def estep(i, carry):
        lbuf[pl.ds(i * 16, 16)] = jnp.exp(lbuf[pl.ds(i * 16, 16)] - mcv)
        return carry

    lax.fori_loop(0, PW // 16, estep, 0)

    for h in range(nh):
        vh_hbm = (v0_hbm, v1_hbm, v2_hbm, v3_hbm)[h]

        def _scale_scatter(j, vrow, h=h):
            def scale(g, carry2):
                ev = lbuf[pl.ds(j * CH + g * 16, 16)]
                for p in range(16):
                    i = g * 16 + p
                    bc = jnp.full((16,), ev[p], jnp.float32)
                    for cc in range(DV // 16):
                        vrow[i, pl.ds(cc * 16, 16)] = (
                            vrow[i, pl.ds(cc * 16, 16)] * bc)
                return carry2

            lax.fori_loop(0, CH // 16, scale, 0)
            if h == 0:
                pltpu.sync_copy(lbuf.at[pl.ds(j * CH, CH)],
                                den_sh.at[dstb.at[j]], add=True)
            pltpu.sync_copy(vrow, agg_sh.at[dstb.at[j]], add=True)

        def _wait_v(vrow, sem, vh_hbm=vh_hbm):
            pltpu.make_async_copy(vh_hbm.at[srcb.at[0]], vrow, sem).wait()

        pltpu.async_copy(vh_hbm.at[srcb.at[0]], va, sema)

        def chunk2pair(t, carry, vh_hbm=vh_hbm):
            j0 = 2 * t
            j1 = 2 * t + 1
            pltpu.async_copy(vh_hbm.at[srcb.at[j1]], vb, semb)
            _wait_v(va, sema)
            _scale_scatter(j0, va)
            pltpu.async_copy(
                vh_hbm.at[srcb.at[jnp.minimum(j0 + 2, NCHUNK - 1)]], va, sema)
            _wait_v(vb, semb)
            _scale_scatter(j1, vb)
            return carry

        lax.fori_loop(0, NCHUNK // 2, chunk2pair, 0)
        _wait_v(va, sema)
        plsc.subcore_barrier()

        for t in range(5):
            pltpu.sync_copy(agg_sh.at[pl.ds(base_row + t * 128, 128)],
                            agg_hbm.at[c, h, pl.ds(base_row + t * 128, 128)])
        if h == 0:
            pltpu.sync_copy(den_sh.at[pl.ds(s * 640, 640)],
                            den_hbm.at[c, pl.ds(s * 640, 640)])
        if h + 1 < nh:
            for t in range(16):
                pltpu.sync_copy(zbuf, agg_sh.at[pl.ds(base_row + t * 40, 40)])
            plsc.subcore_barrier()


def _make_edge(d):
    mesh = plsc.VectorSubcoreMesh(core_axis_name="c", subcore_axis_name="s")
    return pl.kernel(
        functools.partial(_edge_body, d),
        out_type=[
            jax.ShapeDtypeStruct((2, d // DV, NPAD, DV), jnp.float32),
            jax.ShapeDtypeStruct((2, NPAD), jnp.float32),
            jax.ShapeDtypeStruct((2, 1, 16), jnp.float32),
        ],
        mesh=mesh,
        name=f"edge_d{d}",
        compiler_params=pltpu.CompilerParams(
            needs_layout_passes=False, use_tc_tiling_on_sc=False),
        scratch_types=[
            pltpu.VMEM((CH_D[d], d), jnp.bfloat16),
            pltpu.VMEM((CH_D[d], d), jnp.bfloat16),
            pltpu.VMEM((CH_D[d], d), jnp.bfloat16),
            pltpu.VMEM((CH_D[d], d), jnp.bfloat16),
            pltpu.VMEM((CH_D[d], DV), jnp.float32),
            pltpu.VMEM((CH_D[d], DV), jnp.float32),
            pltpu.VMEM((PW,), jnp.float32),
            pltpu.VMEM((PW // CH_D[d], CH_D[d]), jnp.int32),
            pltpu.VMEM((PW // CH_D[d], CH_D[d]), jnp.int32),
            pltpu.VMEM((1, 16), jnp.float32),
            pltpu.VMEM((16, 16), jnp.float32),
            pltpu.VMEM((40, DV), jnp.float32),
            pltpu.VMEM((640,), jnp.float32),
            pltpu.VMEM_SHARED((NPAD, DV), jnp.float32),
            pltpu.VMEM_SHARED((NPAD,), jnp.float32),
            pltpu.VMEM_SHARED((16, 16), jnp.float32),
            pltpu.SemaphoreType.DMA,
            pltpu.SemaphoreType.DMA,
        ],
    )


_EDGE = {d: _make_edge(d) for d in (D1, D2)}


def _edges(qbf, kbf, v, srcr, dstr, d):
    vs = [v[:, h * DV:(h + 1) * DV] for h in range(d // DV)]
    while len(vs) < 4:
        vs.append(vs[0])
    agg_p, den_p, m_p = _EDGE[d](qbf, kbf, *vs, srcr, dstr)
    return agg_p, den_p.reshape(2, NPAD, 1), m_p


def _rescale(m_ref):
    mv = m_ref[...]
    mm = jnp.max(mv)
    sv = jnp.exp(mv - mm)
    return sv[0, 0, 0], sv[1, 0, 0]


def _glue_body(agg_ref, denp_ref, m_ref, s_ref, w_ref, b_ref, o_ref,
               qbf_ref, kbf_ref):
    s0, s1 = _rescale(m_ref)
    agg = (jnp.concatenate([agg_ref[0, h] for h in range(4)], axis=1) * s0
           + jnp.concatenate([agg_ref[1, h] for h in range(4)], axis=1) * s1)
    den = denp_ref[0] * s0 + denp_ref[1] * s1
    h1 = _elu(agg / (den + 1e-16) + s_ref[...])
    res = (
        jnp.dot(h1, w_ref[...], preferred_element_type=jnp.float32) + b_ref[...]
    )
    o_ref[...] = res
    qbf_ref[...] = res[:, :D2].astype(jnp.bfloat16)
    kbf_ref[...] = res[:, D2:2 * D2].astype(jnp.bfloat16)


def _glue(agg_p, den_p, m_p, skip, wcat, bcat):
    n, din = skip.shape
    dout = wcat.shape[1]
    return pl.pallas_call(
        _glue_body,
        grid=(n // MB,),
        in_specs=[
            pl.BlockSpec((2, 4, MB, DV), lambda i: (0, 0, i, 0)),
            pl.BlockSpec((2, MB, 1), lambda i: (0, i, 0)),
            pl.BlockSpec((2, 1, 16), lambda i: (0, 0, 0)),
            pl.BlockSpec((MB, din), lambda i: (i, 0)),
            pl.BlockSpec((din, dout), lambda i: (0, 0)),
            pl.BlockSpec((1, dout), lambda i: (0, 0)),
        ],
        out_specs=[pl.BlockSpec((MB, dout), lambda i: (i, 0)),
                   pl.BlockSpec((MB, D2), lambda i: (i, 0)),
                   pl.BlockSpec((MB, D2), lambda i: (i, 0))],
        out_shape=[jax.ShapeDtypeStruct((n, dout), jnp.float32),
                   jax.ShapeDtypeStruct((n, D2), jnp.bfloat16),
                   jax.ShapeDtypeStruct((n, D2), jnp.bfloat16)],
    )(agg_p, den_p, m_p, skip, wcat, bcat.reshape(1, dout))


def _mlp3(h, w1, b1, w2, b2, w3, b3):
    h = jax.nn.relu(jnp.dot(h, w1, preferred_element_type=jnp.float32) + b1)
    h = jax.nn.relu(jnp.dot(h, w2, preferred_element_type=jnp.float32) + b2)
    return jnp.dot(h, w3, preferred_element_type=jnp.float32) + b3


def _head_body(agg_ref, denp_ref, m_ref, s_ref, wb_ref, bb_ref, wfc_ref,
               bfc_ref, wd1_ref, bd1_ref, wd2_ref, bd2_ref, wd3_ref, bd3_ref,
               wc1_ref, bc1_ref, wc2_ref, bc2_ref, wc3_ref, bc3_ref,
               emb_ref, dom_ref, cls0_ref, cls1_ref):
    s0, s1 = _rescale(m_ref)
    agg = (jnp.concatenate([agg_ref[0, 0], agg_ref[0, 1]], axis=1) * s0
           + jnp.concatenate([agg_ref[1, 0], agg_ref[1, 1]], axis=1) * s1)
    den = denp_ref[0] * s0 + denp_ref[1] * s1
    h2 = _elu(agg / (den + 1e-16) + s_ref[...])
    ss = jnp.dot(h2, wb_ref[...], preferred_element_type=jnp.float32) + bb_ref[...]
    emb = jnp.dot(ss, wfc_ref[...], preferred_element_type=jnp.float32) + bfc_ref[...]
    emb_ref[...] = emb
    m = jnp.max(emb, axis=1, keepdims=True)
    ex = jnp.exp(emb - m)
    p = ex / jnp.sum(ex, axis=1, keepdims=True)
    dom_ref[...] = _mlp3(ss, wd1_ref[...], bd1_ref[...], wd2_ref[...],
                         bd2_ref[...], wd3_ref[...], bd3_ref[...])
    for i, cls_ref in ((0, cls0_ref), (1, cls1_ref)):
        f = p[:, i][:, None] * ss
        cls_ref[...] = _mlp3(f, wc1_ref[i], bc1_ref[i], wc2_ref[i],
                             bc2_ref[i], wc3_ref[i], bc3_ref[i])


def _heads(agg_p, den_p, m_p, s2, Wb, bb, Wfc, bfc, Wd1, bd1, Wd2, bd2,
           Wd3, bd3, Wc1, bc1, Wc2, bc2, Wc3, bc3):
    n = s2.shape[0]
    full = lambda *shape: pl.BlockSpec(shape, lambda i: (0,) * len(shape))
    row = lambda d: pl.BlockSpec((MB, d), lambda i: (i, 0))
    out_shapes = [jax.ShapeDtypeStruct((n, 2), jnp.float32)] * 4
    return pl.pallas_call(
        _head_body,
        grid=(n // MB,),
        in_specs=[
            pl.BlockSpec((2, 2, MB, DV), lambda i: (0, 0, i, 0)),
            pl.BlockSpec((2, MB, 1), lambda i: (0, i, 0)),
            pl.BlockSpec((2, 1, 16), lambda i: (0, 0, 0)),
            row(D2),
            full(D2, 32), full(1, 32), full(32, NC), full(1, NC),
            full(32, 512), full(1, 512), full(512, 512), full(1, 512),
            full(512, 2), full(1, 2),
            full(NC, 32, 512), full(NC, 1, 512),
            full(NC, 512, 512), full(NC, 1, 512),
            full(NC, 512, 2), full(NC, 1, 2),
        ],
        out_specs=[row(2), row(2), row(2), row(2)],
        out_shape=out_shapes,
    )(agg_p, den_p, m_p, s2, Wb, bb.reshape(1, 32), Wfc, bfc.reshape(1, NC),
      Wd1, bd1.reshape(1, 512), Wd2, bd2.reshape(1, 512),
      Wd3, bd3.reshape(1, 2),
      Wc1, bc1.reshape(NC, 1, 512), Wc2, bc2.reshape(NC, 1, 512),
      Wc3, bc3.reshape(NC, 1, 2))


def _pad_edges(ei, d):
    ch = CH_D[d]
    pad = jnp.arange(E_PAD - E, dtype=jnp.int32) % N
    src = jnp.concatenate([ei[0], pad]).reshape(NW, PW // ch, ch)
    dst = jnp.concatenate([ei[1], pad]).reshape(NW, PW // ch, ch)
    return src, dst


def kernel(source_x, target_x, source_edge_index, target_edge_index,
           Wq1, bq1, Wk1, bk1, Wv1, bv1, Ws1, bs1,
           Wq2, bq2, Wk2, bk2, Wv2, bv2, Ws2, bs2,
           Wb, bb, Wfc, bfc,
           Wd1, bd1, Wd2, bd2, Wd3, bd3,
           Wc1, bc1, Wc2, bc2, Wc3, bc3):
    wcat1 = jnp.concatenate([Wq1, Wk1, Wv1, Ws1], axis=1)
    bcat1 = jnp.concatenate([bq1, bk1, bv1, bs1])
    wcat2 = jnp.concatenate([Wq2, Wk2, Wv2, Ws2], axis=1)
    bcat2 = jnp.concatenate([bq2, bk2, bv2, bs2])

    s_srcr1, s_dstr1 = _pad_edges(source_edge_index, D1)
    t_srcr1, t_dstr1 = _pad_edges(target_edge_index, D1)
    s_srcr2, s_dstr2 = _pad_edges(source_edge_index, D2)
    t_srcr2, t_dstr2 = _pad_edges(target_edge_index, D2)
    s_qkvs1, s_qbf1, s_kbf1 = _proj(source_x, wcat1, bcat1, D1)
    t_qkvs1, t_qbf1, t_kbf1 = _proj(target_x, wcat1, bcat1, D1)
    s_agg1, s_den1, s_m1 = _edges(s_qbf1, s_kbf1,
                                  s_qkvs1[:, 2 * D1:3 * D1],
                                  s_srcr1, s_dstr1, D1)
    t_agg1, t_den1, t_m1 = _edges(t_qbf1, t_kbf1,
                                  t_qkvs1[:, 2 * D1:3 * D1],
                                  t_srcr1, t_dstr1, D1)
    s_qkvs2, s_qbf2, s_kbf2 = _glue(s_agg1, s_den1, s_m1,
                                    s_qkvs1[:, 3 * D1:], wcat2, bcat2)
    s_agg2, s_den2, s_m2 = _edges(s_qbf2, s_kbf2,
                                  s_qkvs2[:, 2 * D2:3 * D2],
                                  s_srcr2, s_dstr2, D2)
    t_qkvs2, t_qbf2, t_kbf2 = _glue(t_agg1, t_den1, t_m1,
                                    t_qkvs1[:, 3 * D1:], wcat2, bcat2)
    t_agg2, t_den2, t_m2 = _edges(t_qbf2, t_kbf2,
                                  t_qkvs2[:, 2 * D2:3 * D2],
                                  t_srcr2, t_dstr2, D2)
    s_s2 = s_qkvs2[:, 3 * D2:]
    t_s2 = t_qkvs2[:, 3 * D2:]

    head_w = (Wb, bb, Wfc, bfc, Wd1, bd1, Wd2, bd2, Wd3, bd3,
              Wc1, bc1, Wc2, bc2, Wc3, bc3)
    s_emb, s_dom, s_cls0, s_cls1 = _heads(s_agg2, s_den2, s_m2, s_s2, *head_w)
    _, t_dom, t_cls0, t_cls1 = _heads(t_agg2, t_den2, t_m2, t_s2, *head_w)

    s_out = jnp.stack([s_cls0, s_cls1])
    t_out = jnp.stack([t_cls0, t_cls1])
    return (s_emb, s_dom, t_dom, s_out, t_out)
